# initial kernel scaffold (unmeasured)
import jax
import jax.numpy as jnp
from jax import lax
from jax.experimental import pallas as pl
from jax.experimental.pallas import tpu as pltpu


def kernel(
    x,
):
    def body(*refs):
        pass

    out_shape = jax.ShapeDtypeStruct(..., jnp.float32)
    return pl.pallas_call(body, out_shape=out_shape)(...)


# baseline (device time: 8032 ns/iter reference)
import jax
import jax.numpy as jnp
from jax import lax
from jax.experimental import pallas as pl
from jax.experimental.pallas import tpu as pltpu

N_DEV = 8
BIG = 1e9


def kernel(x):
    m_per, n = x.shape

    def body(x_ref, out_ref, gather_ref, send_sems, recv_sems):
        my_pos = lax.axis_index("i")

        xv = x_ref[:, :]
        vmax = jnp.max(xv, axis=0, keepdims=True)
        row = lax.broadcasted_iota(jnp.int32, (m_per, n), 0).astype(jnp.float32)
        base = (my_pos * m_per).astype(jnp.float32)
        lidx = jnp.min(
            jnp.where(xv == vmax, row + base, BIG), axis=0, keepdims=True
        )
        gather_ref[0, 0:1, :] = vmax
        gather_ref[0, 1:2, :] = lidx

        barrier_sem = pltpu.get_barrier_semaphore()
        for d in range(1, N_DEV):
            peer = lax.rem(my_pos + d, N_DEV)
            pl.semaphore_signal(
                barrier_sem,
                inc=1,
                device_id=(peer,),
                device_id_type=pl.DeviceIdType.MESH,
            )
        pl.semaphore_wait(barrier_sem, N_DEV - 1)

        rdmas = []
        for d in range(1, N_DEV):
            peer = lax.rem(my_pos + d, N_DEV)
            rdma = pltpu.make_async_remote_copy(
                src_ref=gather_ref.at[0],
                dst_ref=gather_ref.at[d],
                send_sem=send_sems.at[d],
                recv_sem=recv_sems.at[d],
                device_id=(peer,),
                device_id_type=pl.DeviceIdType.MESH,
            )
            rdma.start()
            rdmas.append(rdma)
        for rdma in rdmas:
            rdma.wait()

        g = gather_ref[:, :, :]
        vals = g[:, 0, :]
        idxs = g[:, 1, :]
        gmax = jnp.max(vals, axis=0, keepdims=True)
        gidx = jnp.min(
            jnp.where(vals == gmax, idxs, BIG), axis=0, keepdims=True
        )
        out_ref[0:1, :] = gmax
        out_ref[1:2, :] = gidx

    return pl.pallas_call(
        body,
        out_shape=jax.ShapeDtypeStruct((2, n), jnp.float32),
        in_specs=[pl.BlockSpec(memory_space=pltpu.VMEM)],
        out_specs=pl.BlockSpec(memory_space=pltpu.VMEM),
        scratch_shapes=[
            pltpu.VMEM((N_DEV, 2, n), jnp.float32),
            pltpu.SemaphoreType.DMA((N_DEV,)),
            pltpu.SemaphoreType.DMA((N_DEV,)),
        ],
        compiler_params=pltpu.CompilerParams(collective_id=0),
    )(x)


# device time: 7919 ns/iter; 1.0143x vs baseline; 1.0143x over previous
import jax
import jax.numpy as jnp
from jax import lax
from jax.experimental import pallas as pl
from jax.experimental.pallas import tpu as pltpu

N_DEV = 8
BIG = 1e9


def kernel(x):
    m_per, n = x.shape

    def body(x_ref, out_ref, gather_ref, send_sems, recv_sems):
        my_pos = lax.axis_index("i")

        barrier_sem = pltpu.get_barrier_semaphore()
        for d in range(1, N_DEV):
            peer = lax.rem(my_pos + d, N_DEV)
            pl.semaphore_signal(
                barrier_sem,
                inc=1,
                device_id=(peer,),
                device_id_type=pl.DeviceIdType.MESH,
            )

        xv = x_ref[:, :]
        vmax = jnp.max(xv, axis=0, keepdims=True)
        row = lax.broadcasted_iota(jnp.int32, (m_per, n), 0).astype(jnp.float32)
        base = (my_pos * m_per).astype(jnp.float32)
        lidx = jnp.min(
            jnp.where(xv == vmax, row + base, BIG), axis=0, keepdims=True
        )
        gather_ref[0, 0:1, :] = vmax
        gather_ref[0, 1:2, :] = lidx

        pl.semaphore_wait(barrier_sem, N_DEV - 1)

        rdmas = []
        for d in range(1, N_DEV):
            peer = lax.rem(my_pos + d, N_DEV)
            rdma = pltpu.make_async_remote_copy(
                src_ref=gather_ref.at[0],
                dst_ref=gather_ref.at[d],
                send_sem=send_sems.at[d],
                recv_sem=recv_sems.at[d],
                device_id=(peer,),
                device_id_type=pl.DeviceIdType.MESH,
            )
            rdma.start()
            rdmas.append(rdma)
        for rdma in rdmas:
            rdma.wait()

        g = gather_ref[:, :, :]
        vals = g[:, 0, :]
        idxs = g[:, 1, :]
        gmax = jnp.max(vals, axis=0, keepdims=True)
        gidx = jnp.min(
            jnp.where(vals == gmax, idxs, BIG), axis=0, keepdims=True
        )
        out_ref[0:1, :] = gmax
        out_ref[1:2, :] = gidx

    return pl.pallas_call(
        body,
        out_shape=jax.ShapeDtypeStruct((2, n), jnp.float32),
        in_specs=[pl.BlockSpec(memory_space=pltpu.VMEM)],
        out_specs=pl.BlockSpec(memory_space=pltpu.VMEM),
        scratch_shapes=[
            pltpu.VMEM((N_DEV, 2, n), jnp.float32),
            pltpu.SemaphoreType.DMA((N_DEV,)),
            pltpu.SemaphoreType.DMA((N_DEV,)),
        ],
        compiler_params=pltpu.CompilerParams(collective_id=0),
    )(x)
